# baseline (speedup 1.0000x reference)
import jax
import jax.numpy as jnp
from jax import lax
from jax.experimental import pallas as pl
from jax.experimental.pallas import tpu as pltpu

B = 8
N = 2048
TILE = 512
TPB = N // TILE
NT = B * TPB
M = B * N
EPS = 1e-5
C1, C2, C3 = 256, 512, 1024


def _stats_to_scale_shift(acc_s, acc_q, g, be, sc_out, sh_out):
    s = jnp.sum(acc_s[...], axis=1, keepdims=True)
    q = jnp.sum(acc_q[...], axis=1, keepdims=True)
    mean = s * (1.0 / M)
    var = jnp.maximum(q * (1.0 / M) - mean * mean, 0.0)
    sc = g * lax.rsqrt(var + EPS)
    sc_out[...] = sc
    sh_out[...] = be - mean * sc


def _body(xyz_ref, pts_ref, w0a_ref, w0b_ref, w1_ref, w2_ref,
          b0_ref, b1_ref, b2_ref,
          g0_ref, be0_ref, g1_ref, be1_ref, g2_ref, be2_ref,
          out_ref,
          y0s, y1s, a0s, a0q, a1s, a1q, a2s, a2q,
          amax, amin, ymax, ymin, sc0, sh0, sc1, sh1):
    i = pl.program_id(0)
    t = i % NT
    b = t // TPB
    tt = t % TPB

    @pl.when(i < NT)
    def _phase0():
        xv = xyz_ref[t]
        pv = pts_ref[0]
        y = jnp.dot(w0b_ref[...], pv, preferred_element_type=jnp.float32)
        y = y + jnp.dot(w0a_ref[...], xv, preferred_element_type=jnp.float32)
        y = y + b0_ref[...]
        yb = y.astype(jnp.bfloat16)
        y0s[t] = yb
        yf = yb.astype(jnp.float32)

        @pl.when(t == 0)
        def _():
            a0s[...] = yf
            a0q[...] = yf * yf

        @pl.when(t != 0)
        def _():
            a0s[...] += yf
            a0q[...] += yf * yf

        @pl.when(t == NT - 1)
        def _():
            _stats_to_scale_shift(a0s, a0q, g0_ref[...], be0_ref[...], sc0, sh0)

    @pl.when(jnp.logical_and(i >= NT, i < 2 * NT))
    def _phase1():
        y0 = y0s[t].astype(jnp.float32)
        z = jnp.maximum(y0 * sc0[...] + sh0[...], 0.0).astype(jnp.bfloat16)
        y = jnp.dot(w1_ref[...], z, preferred_element_type=jnp.float32)
        y = y + b1_ref[...]
        yb = y.astype(jnp.bfloat16)
        y1s[t] = yb
        yf = yb.astype(jnp.float32)

        @pl.when(t == 0)
        def _():
            a1s[...] = yf
            a1q[...] = yf * yf

        @pl.when(t != 0)
        def _():
            a1s[...] += yf
            a1q[...] += yf * yf

        @pl.when(t == NT - 1)
        def _():
            _stats_to_scale_shift(a1s, a1q, g1_ref[...], be1_ref[...], sc1, sh1)

    @pl.when(i >= 2 * NT)
    def _phase2():
        y1 = y1s[t].astype(jnp.float32)
        z = jnp.maximum(y1 * sc1[...] + sh1[...], 0.0).astype(jnp.bfloat16)
        y = jnp.dot(w2_ref[...], z, preferred_element_type=jnp.float32)
        y = y + b2_ref[...]

        @pl.when(t == 0)
        def _():
            a2s[...] = y
            a2q[...] = y * y

        @pl.when(t != 0)
        def _():
            a2s[...] += y
            a2q[...] += y * y

        @pl.when(tt == 0)
        def _():
            amax[...] = y
            amin[...] = y

        @pl.when(tt != 0)
        def _():
            amax[...] = jnp.maximum(amax[...], y)
            amin[...] = jnp.minimum(amin[...], y)

        @pl.when(tt == TPB - 1)
        def _():
            mx = jnp.max(amax[...], axis=1, keepdims=True)
            mn = jnp.min(amin[...], axis=1, keepdims=True)
            lanes = lax.broadcasted_iota(jnp.int32, (C3, B), 1)
            ymax[...] = jnp.where(lanes == b, mx, ymax[...])
            ymin[...] = jnp.where(lanes == b, mn, ymin[...])

        @pl.when(t == NT - 1)
        def _():
            s = jnp.sum(a2s[...], axis=1, keepdims=True)
            q = jnp.sum(a2q[...], axis=1, keepdims=True)
            mean = s * (1.0 / M)
            var = jnp.maximum(q * (1.0 / M) - mean * mean, 0.0)
            sc = g2_ref[...] * lax.rsqrt(var + EPS)
            sh = be2_ref[...] - mean * sc
            ext = jnp.where(sc >= 0.0, ymax[...], ymin[...])
            out_ref[...] = jnp.maximum(ext * sc + sh, 0.0)


def kernel(xyz, points, W0, b0, g0, beta0, W1, b1, g1, beta1, W2, b2, g2, beta2):
    bf = jnp.bfloat16
    f32 = jnp.float32
    xyz_t = xyz.transpose(1, 0, 2).reshape(3, NT, TILE).transpose(1, 0, 2).astype(bf)
    pts = points.astype(bf)
    w0a = W0[:, :3].astype(bf)
    w0b = W0[:, 3:].astype(bf)
    w1 = W1.astype(bf)
    w2 = W2.astype(bf)

    def col(v):
        return v.reshape(-1, 1).astype(f32)

    grid = 3 * NT
    full = lambda shape: pl.BlockSpec(shape, lambda i: tuple(0 for _ in shape))
    out = pl.pallas_call(
        _body,
        grid=(grid,),
        in_specs=[
            full((NT, 3, TILE)),
            pl.BlockSpec((1, C1, TILE),
                         lambda i: (jnp.minimum(i, NT - 1) // TPB, 0,
                                    jnp.minimum(i, NT - 1) % TPB)),
            full((C1, 3)),
            full((C1, C1)),
            full((C2, C1)),
            full((C3, C2)),
            full((C1, 1)),
            full((C2, 1)),
            full((C3, 1)),
            full((C1, 1)),
            full((C1, 1)),
            full((C2, 1)),
            full((C2, 1)),
            full((C3, 1)),
            full((C3, 1)),
        ],
        out_specs=pl.BlockSpec((C3, B), lambda i: (0, 0)),
        out_shape=jax.ShapeDtypeStruct((C3, B), f32),
        scratch_shapes=[
            pltpu.VMEM((NT, C1, TILE), bf),
            pltpu.VMEM((NT, C2, TILE), bf),
            pltpu.VMEM((C1, TILE), f32),
            pltpu.VMEM((C1, TILE), f32),
            pltpu.VMEM((C2, TILE), f32),
            pltpu.VMEM((C2, TILE), f32),
            pltpu.VMEM((C3, TILE), f32),
            pltpu.VMEM((C3, TILE), f32),
            pltpu.VMEM((C3, TILE), f32),
            pltpu.VMEM((C3, TILE), f32),
            pltpu.VMEM((C3, B), f32),
            pltpu.VMEM((C3, B), f32),
            pltpu.VMEM((C1, 1), f32),
            pltpu.VMEM((C1, 1), f32),
            pltpu.VMEM((C2, 1), f32),
            pltpu.VMEM((C2, 1), f32),
        ],
    )(xyz_t, pts, w0a, w0b, w1, w2,
      col(b0), col(b1), col(b2),
      col(g0), col(beta0), col(g1), col(beta1), col(g2), col(beta2))

    new_points = out.T.reshape(B, C3, 1)
    new_xyz = jnp.zeros((B, 3, 1), f32)
    return new_xyz, new_points
